# Initial kernel scaffold; baseline (speedup 1.0000x reference)
import jax
import jax.numpy as jnp
from jax.experimental import pallas as pl


def kernel(table, encoded_captions):
    raise NotImplementedError("write your pallas kernel here")


# speedup vs baseline: 1.8373x; 1.8373x over previous
import functools

import jax
import jax.numpy as jnp
from jax import lax
from jax.experimental import pallas as pl
from jax.experimental.pallas import tpu as pltpu
from jax.experimental.pallas import tpu_sc as plsc

D = 64
NW = 32
CHUNK = 128


def _gather_kernel(n_chunks, table_hbm, idx_hbm, out_hbm, idx_v, rows_v,
                   gsem0, gsem1):
    wid = lax.axis_index("s") * 2 + lax.axis_index("c")
    base_chunk = wid * n_chunks
    base_row = base_chunk * CHUNK

    pltpu.sync_copy(idx_hbm.at[pl.ds(base_chunk, n_chunks)], idx_v)

    gsems = (gsem0, gsem1)

    for b in range(2):
        pltpu.async_copy(table_hbm.at[idx_v.at[b]], rows_v.at[b], gsems[b])

    def body(c, _):
        for b in range(2):
            cc = 2 * c + b
            pltpu.make_async_copy(
                table_hbm.at[idx_v.at[cc]], rows_v.at[b], gsems[b]
            ).wait()
            pltpu.sync_copy(rows_v.at[b],
                            out_hbm.at[pl.ds(base_row + cc * CHUNK, CHUNK)])
            pltpu.async_copy(
                table_hbm.at[idx_v.at[cc + 2]], rows_v.at[b], gsems[b]
            )
        return _

    lax.fori_loop(0, n_chunks // 2 - 1, body, 0, unroll=False)

    for b in range(2):
        cc = n_chunks - 2 + b
        pltpu.make_async_copy(
            table_hbm.at[idx_v.at[cc]], rows_v.at[b], gsems[b]
        ).wait()
        pltpu.sync_copy(rows_v.at[b],
                        out_hbm.at[pl.ds(base_row + cc * CHUNK, CHUNK)])


def kernel(table, encoded_captions):
    B, S = encoded_captions.shape
    N = B * S
    assert N % (NW * CHUNK * 2) == 0
    n_chunks = N // (NW * CHUNK)
    idx = encoded_captions.reshape(N // CHUNK, CHUNK).astype(jnp.int32)

    mesh = plsc.VectorSubcoreMesh(core_axis_name="c", subcore_axis_name="s")

    run = functools.partial(
        pl.kernel,
        out_type=jax.ShapeDtypeStruct((N, D), jnp.float32),
        mesh=mesh,
        compiler_params=pltpu.CompilerParams(use_tc_tiling_on_sc=False),
        scratch_types=[
            pltpu.VMEM((n_chunks, CHUNK), jnp.int32),
            pltpu.VMEM((2, CHUNK, D), jnp.float32),
            pltpu.SemaphoreType.DMA,
            pltpu.SemaphoreType.DMA,
        ],
    )(functools.partial(_gather_kernel, n_chunks))

    out = run(table, idx)
    return out.reshape(B, S, D)


# speedup vs baseline: 1.8777x; 1.0220x over previous
import functools

import jax
import jax.numpy as jnp
from jax import lax
from jax.experimental import pallas as pl
from jax.experimental.pallas import tpu as pltpu
from jax.experimental.pallas import tpu_sc as plsc

D = 64
NW = 32
CHUNK = 512


def _gather_kernel(n_chunks, table_hbm, idx_hbm, out_hbm, idx_v, rows_v,
                   gsem0, gsem1):
    wid = lax.axis_index("s") * 2 + lax.axis_index("c")
    base_chunk = wid * n_chunks
    base_row = base_chunk * CHUNK

    pltpu.sync_copy(idx_hbm.at[pl.ds(base_chunk, n_chunks)], idx_v)

    gsems = (gsem0, gsem1)

    for b in range(2):
        pltpu.async_copy(table_hbm.at[idx_v.at[b]], rows_v.at[b], gsems[b])

    def body(c, _):
        for b in range(2):
            cc = 2 * c + b
            pltpu.make_async_copy(
                table_hbm.at[idx_v.at[cc]], rows_v.at[b], gsems[b]
            ).wait()
            pltpu.sync_copy(rows_v.at[b],
                            out_hbm.at[pl.ds(base_row + cc * CHUNK, CHUNK)])
            pltpu.async_copy(
                table_hbm.at[idx_v.at[cc + 2]], rows_v.at[b], gsems[b]
            )
        return _

    lax.fori_loop(0, n_chunks // 2 - 1, body, 0, unroll=False)

    for b in range(2):
        cc = n_chunks - 2 + b
        pltpu.make_async_copy(
            table_hbm.at[idx_v.at[cc]], rows_v.at[b], gsems[b]
        ).wait()
        pltpu.sync_copy(rows_v.at[b],
                        out_hbm.at[pl.ds(base_row + cc * CHUNK, CHUNK)])


def kernel(table, encoded_captions):
    B, S = encoded_captions.shape
    N = B * S
    assert N % (NW * CHUNK * 2) == 0
    n_chunks = N // (NW * CHUNK)
    idx = encoded_captions.reshape(N // CHUNK, CHUNK).astype(jnp.int32)

    mesh = plsc.VectorSubcoreMesh(core_axis_name="c", subcore_axis_name="s")

    run = functools.partial(
        pl.kernel,
        out_type=jax.ShapeDtypeStruct((N, D), jnp.float32),
        mesh=mesh,
        compiler_params=pltpu.CompilerParams(use_tc_tiling_on_sc=False),
        scratch_types=[
            pltpu.VMEM((n_chunks, CHUNK), jnp.int32),
            pltpu.VMEM((2, CHUNK, D), jnp.float32),
            pltpu.SemaphoreType.DMA,
            pltpu.SemaphoreType.DMA,
        ],
    )(functools.partial(_gather_kernel, n_chunks))

    out = run(table, idx)
    return out.reshape(B, S, D)
